# baseline (speedup 1.0000x reference)
import jax
import jax.numpy as jnp
from jax.experimental import pallas as pl
from jax.experimental.pallas import tpu as pltpu

_BM = 400
_NBUF = 3


def _s1_body(x_ref, w1_ref, o_ref):
    o_ref[...] = jnp.dot(x_ref[...], w1_ref[...],
                         preferred_element_type=jnp.float32)


def _gcn_body(adj_hbm, s1_hbm, w23_ref,
              mu_ref, lv_ref, bufs, s1_ref, s23_ref, sems, ssem):
    p = pl.program_id(0)
    i = pl.program_id(1)
    nb = pl.num_programs(1)
    g = p * nb + i
    total = 2 * nb

    def issue(step):
        blk = jax.lax.rem(step, nb)
        slot = jax.lax.rem(step, _NBUF)
        pltpu.make_async_copy(
            adj_hbm.at[pl.ds(blk * _BM, _BM), :],
            bufs.at[slot],
            sems.at[slot],
        ).start()

    @pl.when(g == 0)
    def _():
        for s in range(_NBUF - 1):
            issue(jnp.int32(s))
        cp = pltpu.make_async_copy(s1_hbm, s1_ref, ssem)
        cp.start()
        cp.wait()

    @pl.when(g + _NBUF - 1 < total)
    def _():
        issue(g + _NBUF - 1)

    slot = jax.lax.rem(g, _NBUF)
    pltpu.make_async_copy(
        adj_hbm.at[pl.ds(0, _BM), :], bufs.at[slot], sems.at[slot]
    ).wait()

    @pl.when(p == 0)
    def _():
        h1_blk = jnp.maximum(
            jnp.dot(bufs[slot], s1_ref[...],
                    preferred_element_type=jnp.float32), 0.0)
        s23_ref[pl.ds(i * _BM, _BM), :] = jnp.dot(
            h1_blk, w23_ref[...], preferred_element_type=jnp.float32)

    @pl.when(p == 1)
    def _():
        blk = jnp.maximum(
            jnp.dot(bufs[slot], s23_ref[...],
                    preferred_element_type=jnp.float32), 0.0)
        h = mu_ref.shape[1]
        mu_ref[...] = blk[:, :h]
        lv_ref[...] = blk[:, h:]


def kernel(x, adj, W1, W2, W3):
    n, d = x.shape
    h1w = W1.shape[1]
    h2 = W2.shape[1]
    nb = n // _BM
    w23 = jnp.concatenate([W2, W3], axis=1)

    s1 = pl.pallas_call(
        _s1_body,
        out_shape=jax.ShapeDtypeStruct((n, h1w), jnp.float32),
    )(x, W1)

    out_idx = lambda p, i: (i * p, 0)

    mu, logvar = pl.pallas_call(
        _gcn_body,
        grid=(2, nb),
        in_specs=[
            pl.BlockSpec(memory_space=pl.ANY),
            pl.BlockSpec(memory_space=pl.ANY),
            pl.BlockSpec((h1w, 2 * h2), lambda p, i: (0, 0)),
        ],
        out_specs=[
            pl.BlockSpec((_BM, h2), out_idx),
            pl.BlockSpec((_BM, h2), out_idx),
        ],
        out_shape=[
            jax.ShapeDtypeStruct((n, h2), jnp.float32),
            jax.ShapeDtypeStruct((n, h2), jnp.float32),
        ],
        scratch_shapes=[
            pltpu.VMEM((_NBUF, _BM, n), jnp.float32),
            pltpu.VMEM((n, h1w), jnp.float32),
            pltpu.VMEM((n, 2 * h2), jnp.float32),
            pltpu.SemaphoreType.DMA((_NBUF,)),
            pltpu.SemaphoreType.DMA,
        ],
    )(adj, s1, w23)
    return (mu, mu, logvar)


# speedup vs baseline: 1.0476x; 1.0476x over previous
import jax
import jax.numpy as jnp
from jax.experimental import pallas as pl
from jax.experimental.pallas import tpu as pltpu

_BM = 400


def _gcn_body(adj_ref, x_ref, w1_ref, w23_ref,
              mu_ref, lv_ref, s1_ref, s23_ref):
    p = pl.program_id(0)
    i = pl.program_id(1)

    @pl.when(jnp.logical_and(p == 0, i == 0))
    def _():
        s1_ref[...] = jnp.dot(x_ref[...], w1_ref[...],
                              preferred_element_type=jnp.float32)

    @pl.when(p == 0)
    def _():
        h1_blk = jnp.maximum(
            jnp.dot(adj_ref[...], s1_ref[...],
                    preferred_element_type=jnp.float32), 0.0)
        s23_ref[pl.ds(i * _BM, _BM), :] = jnp.dot(
            h1_blk, w23_ref[...], preferred_element_type=jnp.float32)

    @pl.when(p == 1)
    def _():
        blk = jnp.maximum(
            jnp.dot(adj_ref[...], s23_ref[...],
                    preferred_element_type=jnp.float32), 0.0)
        h = mu_ref.shape[1]
        mu_ref[...] = blk[:, :h]
        lv_ref[...] = blk[:, h:]


def kernel(x, adj, W1, W2, W3):
    n, d = x.shape
    h1w = W1.shape[1]
    h2 = W2.shape[1]
    nb = n // _BM
    w23 = jnp.concatenate([W2, W3], axis=1)

    out_idx = lambda p, i: (i * p, 0)

    mu, logvar = pl.pallas_call(
        _gcn_body,
        grid=(2, nb),
        in_specs=[
            pl.BlockSpec((_BM, n), lambda p, i: (i, 0)),
            pl.BlockSpec((n, d), lambda p, i: (0, 0)),
            pl.BlockSpec((d, h1w), lambda p, i: (0, 0)),
            pl.BlockSpec((h1w, 2 * h2), lambda p, i: (0, 0)),
        ],
        out_specs=[
            pl.BlockSpec((_BM, h2), out_idx),
            pl.BlockSpec((_BM, h2), out_idx),
        ],
        out_shape=[
            jax.ShapeDtypeStruct((n, h2), jnp.float32),
            jax.ShapeDtypeStruct((n, h2), jnp.float32),
        ],
        scratch_shapes=[
            pltpu.VMEM((n, h1w), jnp.float32),
            pltpu.VMEM((n, 2 * h2), jnp.float32),
        ],
    )(adj, x, W1, w23)
    return (mu, mu, logvar)


# speedup vs baseline: 1.1849x; 1.1311x over previous
import jax
import jax.numpy as jnp
from jax.experimental import pallas as pl
from jax.experimental.pallas import tpu as pltpu

_BI = 2048


def _s1_body(x_ref, w1_ref, o_ref):
    o_ref[...] = jnp.dot(x_ref[...], w1_ref[...],
                         preferred_element_type=jnp.float32)


def kernel(x, adj, W1, W2, W3):
    n, d = x.shape
    h1w = W1.shape[1]
    h2 = W2.shape[1]
    nb = -(-n // _BI)
    tail = n - (nb - 1) * _BI
    npad = nb * _BI
    w23 = jnp.concatenate([W2, W3], axis=1)

    s1 = pl.pallas_call(
        _s1_body,
        out_shape=jax.ShapeDtypeStruct((n, h1w), jnp.float32),
    )(x, W1)

    def body(adj_ref, s1_hbm, w23_ref, mu_hbm, lv_hbm,
             s1_ref, s23_ref, po_ref, acc_ref, mus_ref, lvs_ref,
             ssem, osem1, osem2):
        p = pl.program_id(0)
        ir = pl.program_id(1)
        jj = pl.program_id(2)

        j0 = jax.lax.rem(ir + 1 + jj, nb)
        i1 = jnp.minimum(ir, nb - 2)
        j1 = jnp.minimum(i1 + 1 + jj, nb - 1)
        i = jnp.where(p == 0, ir, i1)
        j = jnp.where(p == 0, j0, j1)
        rows = pl.ds(i * _BI, _BI)

        @pl.when(jnp.logical_and(p == 0,
                                 jnp.logical_and(ir == 0, jj == 0)))
        def _():
            cp = pltpu.make_async_copy(
                s1_hbm, s1_ref.at[pl.ds(0, n), :], ssem)
            cp.start()
            cp.wait()
            if npad > n:
                s1_ref[pl.ds(n, npad - n), :] = jnp.zeros(
                    (npad - n, h1w), jnp.float32)

        if tail < _BI:
            @pl.when(j == nb - 1)
            def _():
                adj_ref[:, pl.ds(tail, _BI - tail)] = jnp.zeros(
                    (_BI, _BI - tail), jnp.float32)

        def emit_full(row_idx):
            final = jnp.maximum(po_ref[pl.ds(row_idx * _BI, _BI), :], 0.0)
            mus_ref[...] = final[:, :h2]
            lvs_ref[...] = final[:, h2:]
            c1 = pltpu.make_async_copy(
                mus_ref, mu_hbm.at[pl.ds(row_idx * _BI, _BI), :], osem1)
            c2 = pltpu.make_async_copy(
                lvs_ref, lv_hbm.at[pl.ds(row_idx * _BI, _BI), :], osem2)
            c1.start()
            c2.start()
            c1.wait()
            c2.wait()

        def emit_last():
            base = (nb - 1) * _BI
            final = jnp.maximum(po_ref[pl.ds(base, _BI), :], 0.0)
            mus_ref[...] = final[:, :h2]
            lvs_ref[...] = final[:, h2:]
            c1 = pltpu.make_async_copy(
                mus_ref.at[pl.ds(0, tail), :],
                mu_hbm.at[pl.ds(base, tail), :], osem1)
            c2 = pltpu.make_async_copy(
                lvs_ref.at[pl.ds(0, tail), :],
                lv_hbm.at[pl.ds(base, tail), :], osem2)
            c1.start()
            c2.start()
            c1.wait()
            c2.wait()

        @pl.when(p == 0)
        def _():
            contrib = jnp.dot(
                adj_ref[...], s1_ref[pl.ds(j * _BI, _BI), :],
                preferred_element_type=jnp.float32)
            acc_ref[...] = jnp.where(jj == 0, contrib,
                                     acc_ref[...] + contrib)

            @pl.when(jj == 0)
            def _():
                po_ref[rows, :] = jnp.zeros((_BI, 2 * h2), jnp.float32)

            @pl.when(j < i)
            def _():
                po_ref[rows, :] += jnp.dot(
                    adj_ref[...], s23_ref[pl.ds(j * _BI, _BI), :],
                    preferred_element_type=jnp.float32)

            @pl.when(jj == nb - 1)
            def _():
                h1_blk = jnp.maximum(acc_ref[...], 0.0)
                s23_blk = jnp.dot(h1_blk, w23_ref[...],
                                  preferred_element_type=jnp.float32)
                s23_ref[rows, :] = s23_blk

                @pl.when(i == nb - 1)
                def _():
                    if npad > n:
                        s23_ref[pl.ds(n, npad - n), :] = jnp.zeros(
                            (npad - n, 2 * h2), jnp.float32)

                po_ref[rows, :] += jnp.dot(
                    adj_ref[...], s23_ref[pl.ds(i * _BI, _BI), :],
                    preferred_element_type=jnp.float32)

                @pl.when(i == nb - 1)
                def _():
                    emit_last()

        @pl.when(p == 1)
        def _():
            valid = jnp.logical_and(ir <= nb - 2, i1 + 1 + jj <= nb - 1)

            @pl.when(valid)
            def _():
                po_ref[rows, :] += jnp.dot(
                    adj_ref[...], s23_ref[pl.ds(j * _BI, _BI), :],
                    preferred_element_type=jnp.float32)

                @pl.when(j == nb - 1)
                def _():
                    emit_full(i)

    def adj_idx(p, ir, jj):
        j0 = jax.lax.rem(ir + 1 + jj, nb)
        i1 = jnp.minimum(ir, nb - 2)
        j1 = jnp.minimum(i1 + 1 + jj, nb - 1)
        return (jnp.where(p == 0, ir, i1), jnp.where(p == 0, j0, j1))

    mu, logvar = pl.pallas_call(
        body,
        grid=(2, nb, nb),
        in_specs=[
            pl.BlockSpec((_BI, _BI), adj_idx),
            pl.BlockSpec(memory_space=pl.ANY),
            pl.BlockSpec((h1w, 2 * h2), lambda p, i, j: (0, 0)),
        ],
        out_specs=[
            pl.BlockSpec(memory_space=pl.ANY),
            pl.BlockSpec(memory_space=pl.ANY),
        ],
        out_shape=[
            jax.ShapeDtypeStruct((n, h2), jnp.float32),
            jax.ShapeDtypeStruct((n, h2), jnp.float32),
        ],
        scratch_shapes=[
            pltpu.VMEM((npad, h1w), jnp.float32),
            pltpu.VMEM((npad, 2 * h2), jnp.float32),
            pltpu.VMEM((npad, 2 * h2), jnp.float32),
            pltpu.VMEM((_BI, h1w), jnp.float32),
            pltpu.VMEM((_BI, h2), jnp.float32),
            pltpu.VMEM((_BI, h2), jnp.float32),
            pltpu.SemaphoreType.DMA,
            pltpu.SemaphoreType.DMA,
            pltpu.SemaphoreType.DMA,
        ],
    )(adj, s1, w23)
    return (mu, mu, logvar)


# speedup vs baseline: 1.1891x; 1.0036x over previous
import jax
import jax.numpy as jnp
from jax.experimental import pallas as pl
from jax.experimental.pallas import tpu as pltpu

_BI = 2048


def _s1_body(x_ref, w1_ref, o_ref):
    o_ref[...] = jnp.dot(x_ref[...], w1_ref[...],
                         preferred_element_type=jnp.float32
                         ).astype(jnp.bfloat16)


def kernel(x, adj, W1, W2, W3):
    n, d = x.shape
    h1w = W1.shape[1]
    h2 = W2.shape[1]
    nb = -(-n // _BI)
    tail = n - (nb - 1) * _BI
    npad = nb * _BI
    w23 = jnp.concatenate([W2, W3], axis=1)

    s1 = pl.pallas_call(
        _s1_body,
        out_shape=jax.ShapeDtypeStruct((n, h1w), jnp.bfloat16),
    )(x, W1)

    def body(adj_ref, s1_hbm, w23_ref, mu_hbm, lv_hbm,
             s1_ref, s23_ref, po_ref, acc_ref, mus_ref, lvs_ref,
             ssem, osem1, osem2):
        p = pl.program_id(0)
        ir = pl.program_id(1)
        jj = pl.program_id(2)

        j0 = jax.lax.rem(ir + 1 + jj, nb)
        i1 = jnp.minimum(ir, nb - 2)
        j1 = jnp.minimum(i1 + 1 + jj, nb - 1)
        i = jnp.where(p == 0, ir, i1)
        j = jnp.where(p == 0, j0, j1)
        rows = pl.ds(i * _BI, _BI)

        @pl.when(jnp.logical_and(p == 0,
                                 jnp.logical_and(ir == 0, jj == 0)))
        def _():
            cp = pltpu.make_async_copy(
                s1_hbm, s1_ref.at[pl.ds(0, n), :], ssem)
            cp.start()
            cp.wait()
            if npad > n:
                s1_ref[pl.ds(n, npad - n), :] = jnp.zeros(
                    (npad - n, h1w), jnp.bfloat16)

        if tail < _BI:
            @pl.when(j == nb - 1)
            def _():
                adj_ref[:, pl.ds(tail, _BI - tail)] = jnp.zeros(
                    (_BI, _BI - tail), jnp.float32)

        def emit_full(row_idx):
            final = jnp.maximum(po_ref[pl.ds(row_idx * _BI, _BI), :], 0.0)
            mus_ref[...] = final[:, :h2]
            lvs_ref[...] = final[:, h2:]
            c1 = pltpu.make_async_copy(
                mus_ref, mu_hbm.at[pl.ds(row_idx * _BI, _BI), :], osem1)
            c2 = pltpu.make_async_copy(
                lvs_ref, lv_hbm.at[pl.ds(row_idx * _BI, _BI), :], osem2)
            c1.start()
            c2.start()
            c1.wait()
            c2.wait()

        def emit_last():
            base = (nb - 1) * _BI
            final = jnp.maximum(po_ref[pl.ds(base, _BI), :], 0.0)
            mus_ref[...] = final[:, :h2]
            lvs_ref[...] = final[:, h2:]
            c1 = pltpu.make_async_copy(
                mus_ref.at[pl.ds(0, tail), :],
                mu_hbm.at[pl.ds(base, tail), :], osem1)
            c2 = pltpu.make_async_copy(
                lvs_ref.at[pl.ds(0, tail), :],
                lv_hbm.at[pl.ds(base, tail), :], osem2)
            c1.start()
            c2.start()
            c1.wait()
            c2.wait()

        @pl.when(p == 0)
        def _():
            tb = adj_ref[...].astype(jnp.bfloat16)
            contrib = jnp.dot(
                tb, s1_ref[pl.ds(j * _BI, _BI), :],
                preferred_element_type=jnp.float32)
            acc_ref[...] = jnp.where(jj == 0, contrib,
                                     acc_ref[...] + contrib)

            @pl.when(jj == 0)
            def _():
                po_ref[rows, :] = jnp.zeros((_BI, 2 * h2), jnp.float32)

            @pl.when(j < i)
            def _():
                po_ref[rows, :] += jnp.dot(
                    tb, s23_ref[pl.ds(j * _BI, _BI), :],
                    preferred_element_type=jnp.float32)

            @pl.when(jj == nb - 1)
            def _():
                h1_blk = jnp.maximum(acc_ref[...], 0.0)
                s23_blk = jnp.dot(h1_blk, w23_ref[...],
                                  preferred_element_type=jnp.float32)
                s23_ref[rows, :] = s23_blk.astype(jnp.bfloat16)

                @pl.when(i == nb - 1)
                def _():
                    if npad > n:
                        s23_ref[pl.ds(n, npad - n), :] = jnp.zeros(
                            (npad - n, 2 * h2), jnp.bfloat16)

                po_ref[rows, :] += jnp.dot(
                    tb, s23_ref[pl.ds(i * _BI, _BI), :],
                    preferred_element_type=jnp.float32)

                @pl.when(i == nb - 1)
                def _():
                    emit_last()

        @pl.when(p == 1)
        def _():
            valid = jnp.logical_and(ir <= nb - 2, i1 + 1 + jj <= nb - 1)

            @pl.when(valid)
            def _():
                po_ref[rows, :] += jnp.dot(
                    adj_ref[...].astype(jnp.bfloat16),
                    s23_ref[pl.ds(j * _BI, _BI), :],
                    preferred_element_type=jnp.float32)

                @pl.when(j == nb - 1)
                def _():
                    emit_full(i)

    def adj_idx(p, ir, jj):
        j0 = jax.lax.rem(ir + 1 + jj, nb)
        i1 = jnp.minimum(ir, nb - 2)
        j1 = jnp.minimum(i1 + 1 + jj, nb - 1)
        return (jnp.where(p == 0, ir, i1), jnp.where(p == 0, j0, j1))

    mu, logvar = pl.pallas_call(
        body,
        grid=(2, nb, nb),
        in_specs=[
            pl.BlockSpec((_BI, _BI), adj_idx),
            pl.BlockSpec(memory_space=pl.ANY),
            pl.BlockSpec((h1w, 2 * h2), lambda p, i, j: (0, 0)),
        ],
        out_specs=[
            pl.BlockSpec(memory_space=pl.ANY),
            pl.BlockSpec(memory_space=pl.ANY),
        ],
        out_shape=[
            jax.ShapeDtypeStruct((n, h2), jnp.float32),
            jax.ShapeDtypeStruct((n, h2), jnp.float32),
        ],
        scratch_shapes=[
            pltpu.VMEM((npad, h1w), jnp.bfloat16),
            pltpu.VMEM((npad, 2 * h2), jnp.bfloat16),
            pltpu.VMEM((npad, 2 * h2), jnp.float32),
            pltpu.VMEM((_BI, h1w), jnp.float32),
            pltpu.VMEM((_BI, h2), jnp.float32),
            pltpu.VMEM((_BI, h2), jnp.float32),
            pltpu.SemaphoreType.DMA,
            pltpu.SemaphoreType.DMA,
            pltpu.SemaphoreType.DMA,
        ],
    )(adj, s1, w23)
    return (mu, mu, logvar)
